# Initial kernel scaffold; baseline (speedup 1.0000x reference)
import jax
import jax.numpy as jnp
from jax.experimental import pallas as pl


def kernel(x, Wq, Wk, Wv, Wo, gate_w, Wg, Wu, Wd, norm1, norm2, norm3):
    raise NotImplementedError("write your pallas kernel here")


# speedup vs baseline: 1.3815x; 1.3815x over previous
import functools

import jax
import jax.numpy as jnp
from jax.experimental import pallas as pl

B, S, D = 1, 2048, 1024
NH, HD = 16, 64
E, K, F = 8, 2, 512
EPS = 1e-6

S_BLK = 256
S_BLKS = S // S_BLK
Q_BLK = 512
Q_BLKS = S // Q_BLK
NEG = -1e30


def _rms(xf, w):
    var = jnp.mean(xf * xf, axis=-1, keepdims=True)
    return xf * jax.lax.rsqrt(var + EPS) * w


def _a1_body(x_ref, n1_ref, wq_ref, wk_ref, wv_ref, q_ref, k_ref, v_ref):
    xn = _rms(x_ref[...], n1_ref[...])
    q_ref[...] = jnp.dot(xn, wq_ref[...], preferred_element_type=jnp.float32)
    k_ref[...] = jnp.dot(xn, wk_ref[...], preferred_element_type=jnp.float32)
    v_ref[...] = jnp.dot(xn, wv_ref[...], preferred_element_type=jnp.float32)


def _one_head(q, k, v):
    s = jax.lax.dot_general(q, k, (((1,), (1,)), ((), ())),
                            preferred_element_type=jnp.float32)
    m = jnp.max(s, axis=1, keepdims=True)
    p = jnp.exp(s - m)
    l = jnp.sum(p, axis=1, keepdims=True)
    return jnp.dot(p / l, v, preferred_element_type=jnp.float32)


def _a2_body(q_ref, k_ref, v_ref, o_ref):
    q = q_ref[...] * (1.0 / 8.0)
    k = k_ref[...]
    v = v_ref[...]
    oa = _one_head(q[:, :HD], k[:, :HD], v[:, :HD])
    ob = _one_head(q[:, HD:], k[:, HD:], v[:, HD:])
    o_ref[...] = jnp.concatenate([oa, ob], axis=1)


def _a3_body(attn_ref, x_ref, wo_ref, n2_ref, gw_ref,
             x1_ref, hx2_ref, wfull_ref, stats_ref):
    x1 = x_ref[...] + jnp.dot(attn_ref[...], wo_ref[...],
                              preferred_element_type=jnp.float32)
    x1_ref[...] = x1
    hx2 = _rms(x1, n2_ref[...])
    hx2_ref[...] = hx2
    logits = jnp.dot(hx2, gw_ref[...], preferred_element_type=jnp.float32)
    eids = jax.lax.broadcasted_iota(jnp.int32, (S_BLK, E), 1)
    m1 = jnp.max(logits, axis=1, keepdims=True)
    e1 = jnp.min(jnp.where(logits == m1, eids, E), axis=1, keepdims=True)
    l2 = jnp.where(eids == e1, NEG, logits)
    m2 = jnp.max(l2, axis=1, keepdims=True)
    e2 = jnp.min(jnp.where(l2 == m2, eids, E), axis=1, keepdims=True)
    r = jnp.exp(m2 - m1)
    w1 = 1.0 / (1.0 + r)
    w2 = 1.0 - w1
    wfull = jnp.where(eids == e1, w1, 0.0) + jnp.where(eids == e2, w2, 0.0)
    wfull_ref[...] = wfull
    p = jnp.exp(logits - m1)
    probs = p / jnp.sum(p, axis=1, keepdims=True)
    psum = jnp.sum(probs, axis=0, keepdims=True)
    cnt = jnp.sum((wfull > 0.0).astype(jnp.float32), axis=0, keepdims=True)
    pad = jnp.zeros((1, 128 - 2 * E), jnp.float32)
    stats_ref[...] = jnp.concatenate([psum, cnt, pad], axis=1).reshape(1, 1, 128)


def _d_body(hx2_ref, x1_ref, wfull_ref, wg_ref, wu_ref, wd_ref, y_ref):
    e = pl.program_id(1)
    hx2 = hx2_ref[...]
    g = jnp.dot(hx2, wg_ref[0], preferred_element_type=jnp.float32)
    u = jnp.dot(hx2, wu_ref[0], preferred_element_type=jnp.float32)
    h = g * (1.0 / (1.0 + jnp.exp(-g))) * u
    o = jnp.dot(h, wd_ref[0], preferred_element_type=jnp.float32)
    eids = jax.lax.broadcasted_iota(jnp.int32, (S_BLK, E), 1)
    wcol = jnp.sum(jnp.where(eids == e, wfull_ref[...], 0.0), axis=1,
                   keepdims=True)
    contrib = o * wcol

    @pl.when(e == 0)
    def _init():
        y_ref[...] = x1_ref[...] + contrib

    @pl.when(e != 0)
    def _acc():
        y_ref[...] += contrib


def _c_body(y_ref, n3_ref, stats_ref, xo_ref, aux_ref):
    xo_ref[...] = _rms(y_ref[...], n3_ref[...])

    @pl.when(pl.program_id(0) == 0)
    def _aux():
        st = stats_ref[...].reshape(S_BLKS, 128)
        psum = st[:, 0:E]
        cnt = st[:, E:2 * E]
        a = jnp.sum(jnp.sum(psum, axis=0) * jnp.sum(cnt, axis=0)) / (S * S)
        aux_ref[...] = jnp.full((8, 128), a, jnp.float32)


def kernel(x, Wq, Wk, Wv, Wo, gate_w, Wg, Wu, Wd, norm1, norm2, norm3):
    xf = x.reshape(S, D)
    n1 = norm1.reshape(1, D)
    n2 = norm2.reshape(1, D)
    n3 = norm3.reshape(1, D)

    full = lambda shp: pl.BlockSpec(shp, lambda *_: tuple(0 for _ in shp))

    q, k, v = pl.pallas_call(
        _a1_body,
        grid=(S_BLKS,),
        in_specs=[pl.BlockSpec((S_BLK, D), lambda i: (i, 0)),
                  full((1, D)), full((D, D)), full((D, D)), full((D, D))],
        out_specs=[pl.BlockSpec((S_BLK, D), lambda i: (i, 0))] * 3,
        out_shape=[jax.ShapeDtypeStruct((S, D), jnp.float32)] * 3,
    )(xf, n1, Wq, Wk, Wv)

    attn = pl.pallas_call(
        _a2_body,
        grid=(NH // 2, Q_BLKS),
        in_specs=[pl.BlockSpec((Q_BLK, 2 * HD), lambda h, i: (i, h)),
                  pl.BlockSpec((S, 2 * HD), lambda h, i: (0, h)),
                  pl.BlockSpec((S, 2 * HD), lambda h, i: (0, h))],
        out_specs=pl.BlockSpec((Q_BLK, 2 * HD), lambda h, i: (i, h)),
        out_shape=jax.ShapeDtypeStruct((S, D), jnp.float32),
    )(q, k, v)

    x1, hx2, wfull, stats = pl.pallas_call(
        _a3_body,
        grid=(S_BLKS,),
        in_specs=[pl.BlockSpec((S_BLK, D), lambda i: (i, 0)),
                  pl.BlockSpec((S_BLK, D), lambda i: (i, 0)),
                  full((D, D)), full((1, D)), full((D, E))],
        out_specs=[pl.BlockSpec((S_BLK, D), lambda i: (i, 0)),
                   pl.BlockSpec((S_BLK, D), lambda i: (i, 0)),
                   pl.BlockSpec((S_BLK, E), lambda i: (i, 0)),
                   pl.BlockSpec((1, 1, 128), lambda i: (i, 0, 0))],
        out_shape=[jax.ShapeDtypeStruct((S, D), jnp.float32),
                   jax.ShapeDtypeStruct((S, D), jnp.float32),
                   jax.ShapeDtypeStruct((S, E), jnp.float32),
                   jax.ShapeDtypeStruct((S_BLKS, 1, 128), jnp.float32)],
    )(attn, xf, Wo, n2, gate_w)

    y = pl.pallas_call(
        _d_body,
        grid=(S_BLKS, E),
        in_specs=[pl.BlockSpec((S_BLK, D), lambda i, e: (i, 0)),
                  pl.BlockSpec((S_BLK, D), lambda i, e: (i, 0)),
                  pl.BlockSpec((S_BLK, E), lambda i, e: (i, 0)),
                  pl.BlockSpec((1, D, F), lambda i, e: (e, 0, 0)),
                  pl.BlockSpec((1, D, F), lambda i, e: (e, 0, 0)),
                  pl.BlockSpec((1, F, D), lambda i, e: (e, 0, 0))],
        out_specs=pl.BlockSpec((S_BLK, D), lambda i, e: (i, 0)),
        out_shape=jax.ShapeDtypeStruct((S, D), jnp.float32),
    )(hx2, x1, wfull, Wg, Wu, Wd)

    xo, aux = pl.pallas_call(
        _c_body,
        grid=(S_BLKS,),
        in_specs=[pl.BlockSpec((S_BLK, D), lambda i: (i, 0)),
                  full((1, D)),
                  pl.BlockSpec((S_BLKS, 1, 128), lambda i: (0, 0, 0))],
        out_specs=[pl.BlockSpec((S_BLK, D), lambda i: (i, 0)),
                   pl.BlockSpec((8, 128), lambda i: (0, 0))],
        out_shape=[jax.ShapeDtypeStruct((S, D), jnp.float32),
                   jax.ShapeDtypeStruct((8, 128), jnp.float32)],
    )(y, n3, stats)

    return xo.reshape(B, S, D), aux[0, 0]


# speedup vs baseline: 1.5822x; 1.1453x over previous
import functools

import jax
import jax.numpy as jnp
from jax import lax
from jax.experimental import pallas as pl
from jax.experimental.pallas import tpu as pltpu
from jax.experimental.pallas import tpu_sc as plsc

B, S, D = 1, 2048, 1024
NH, HD = 16, 64
E, K, F = 8, 2, 512
EPS = 1e-6

S_BLK = 256
S_BLKS = S // S_BLK
Q_BLK = 512
Q_BLKS = S // Q_BLK
NEG = -1e30

NC, NS, L = 2, 16, 16
NW = NC * NS
TPW = S // NW
TS = 256
NT = (S * K) // TS + E
NTP = 32
P = NT * TS


def _rms(xf, w):
    var = jnp.mean(xf * xf, axis=-1, keepdims=True)
    return xf * jax.lax.rsqrt(var + EPS) * w


def _a1_body(x_ref, n1_ref, wq_ref, wk_ref, wv_ref, q_ref, k_ref, v_ref):
    xn = _rms(x_ref[...], n1_ref[...])
    q_ref[...] = jnp.dot(xn, wq_ref[...], preferred_element_type=jnp.float32)
    k_ref[...] = jnp.dot(xn, wk_ref[...], preferred_element_type=jnp.float32)
    v_ref[...] = jnp.dot(xn, wv_ref[...], preferred_element_type=jnp.float32)


def _one_head(q, k, v):
    s = jax.lax.dot_general(q, k, (((1,), (1,)), ((), ())),
                            preferred_element_type=jnp.float32)
    m = jnp.max(s, axis=1, keepdims=True)
    p = jnp.exp(s - m)
    l = jnp.sum(p, axis=1, keepdims=True)
    return jnp.dot(p / l, v, preferred_element_type=jnp.float32)


def _a2_body(q_ref, k_ref, v_ref, o_ref):
    q = q_ref[...] * (1.0 / 8.0)
    k = k_ref[...]
    v = v_ref[...]
    oa = _one_head(q[:, :HD], k[:, :HD], v[:, :HD])
    ob = _one_head(q[:, HD:], k[:, HD:], v[:, HD:])
    o_ref[...] = jnp.concatenate([oa, ob], axis=1)


def _a3_body(attn_ref, x_ref, wo_ref, n2_ref, gw_ref, x1_ref, hx2_ref,
             logt_ref):
    x1 = x_ref[...] + jnp.dot(attn_ref[...], wo_ref[...],
                              preferred_element_type=jnp.float32)
    x1_ref[...] = x1
    hx2 = _rms(x1, n2_ref[...])
    hx2_ref[...] = hx2
    logt_ref[...] = jax.lax.dot_general(
        gw_ref[...], hx2, (((0,), (1,)), ((), ())),
        preferred_element_type=jnp.float32)


def _sc1a_body(logt, epair, wpair, lcg, psg, lv, e1s, e2s, w1s, w2s, ovi, ovf):
    wid = lax.axis_index("s") * NC + lax.axis_index("c")
    base = wid * TPW
    for e in range(E):
        pltpu.sync_copy(logt.at[e, pl.ds(base, TPW)],
                        lv.at[pl.ds(e * TPW, TPW)])
    lane = lax.broadcasted_iota(jnp.int32, (L,), 0)
    lc = jnp.zeros((L,), jnp.int32)
    ps = jnp.zeros((L,), jnp.float32)
    for c in range(TPW // L):
        les = [lv[pl.ds(e * TPW + c * L, L)] for e in range(E)]
        m1 = jnp.full((L,), NEG, jnp.float32)
        m2 = jnp.full((L,), NEG, jnp.float32)
        e1 = jnp.zeros((L,), jnp.int32)
        e2 = jnp.zeros((L,), jnp.int32)
        for e in range(E):
            le = les[e]
            gt1 = le > m1
            gt2 = jnp.logical_and(le > m2, jnp.logical_not(gt1))
            e2 = jnp.where(gt1, e1, jnp.where(gt2, e, e2))
            m2 = jnp.where(gt1, m1, jnp.where(gt2, le, m2))
            e1 = jnp.where(gt1, e, e1)
            m1 = jnp.where(gt1, le, m1)
        r = jnp.exp(m2 - m1)
        w1 = 1.0 / (1.0 + r)
        pes = [jnp.exp(les[e] - m1) for e in range(E)]
        tot = pes[0]
        for e in range(1, E):
            tot = tot + pes[e]
        inv = 1.0 / tot
        for e in range(E):
            ps = ps + jnp.where(lane == e, jnp.sum(pes[e] * inv), 0.0)
            ce = (jnp.sum((e1 == e).astype(jnp.int32))
                  + jnp.sum((e2 == e).astype(jnp.int32)))
            lc = lc + jnp.where(lane == e, ce, 0)
        e1s[pl.ds(c * L, L)] = e1
        e2s[pl.ds(c * L, L)] = e2
        w1s[pl.ds(c * L, L)] = w1
        w2s[pl.ds(c * L, L)] = 1.0 - w1
    ovi[...] = lc
    ovf[...] = ps
    pltpu.sync_copy(e1s, epair.at[pl.ds(base, TPW)])
    pltpu.sync_copy(e2s, epair.at[pl.ds(S + base, TPW)])
    pltpu.sync_copy(w1s, wpair.at[pl.ds(base, TPW)])
    pltpu.sync_copy(w2s, wpair.at[pl.ds(S + base, TPW)])
    pltpu.sync_copy(ovi, lcg.at[pl.ds(wid * L, L)])
    pltpu.sync_copy(ovf, psg.at[pl.ds(wid * L, L)])


def _sc1b_body(epair, lcg, psg, hx2, xs, slots, tmeta, aux,
               lcv, psv, e1s, e2s, s1v, s2v, rows, tmv, auxs, sem):
    wid = lax.axis_index("s") * NC + lax.axis_index("c")
    base = wid * TPW
    pltpu.sync_copy(lcg, lcv)
    pltpu.sync_copy(epair.at[pl.ds(base, TPW)], e1s)
    pltpu.sync_copy(epair.at[pl.ds(S + base, TPW)], e2s)
    pltpu.sync_copy(hx2.at[pl.ds(base, TPW)], rows)
    lane = lax.broadcasted_iota(jnp.int32, (L,), 0)
    zi = jnp.zeros((L,), jnp.int32)

    def _acc(w, carry):
        t, p = carry
        v = lcv[pl.ds(w * L, L)]
        return t + v, p + v * (w < wid).astype(jnp.int32)

    tot, pref = lax.fori_loop(0, NW, _acc, (zi, zi))
    nt = (tot + (TS - 1)) >> 8
    tb = plsc.cumsum(nt) - nt
    r_vec = tb * TS + pref
    for c in range(TPW // L):
        for (esrc, sdst) in ((e1s, s1v), (e2s, s2v)):
            ev = esrc[pl.ds(c * L, L)]
            slot = zi
            for e in range(E):
                m = ev == e
                mi = m.astype(jnp.int32)
                cs = plsc.cumsum(mi)
                re = jnp.sum(jnp.where(lane == e, r_vec, 0))
                slot = jnp.where(m, re + cs - 1, slot)
                r_vec = r_vec + jnp.where(lane == e, jnp.sum(mi), 0)
            sdst[pl.ds(c * L, L)] = slot
    pltpu.async_copy(rows, xs.at[s1v], sem).wait()
    pltpu.async_copy(rows, xs.at[s2v], sem).wait()
    pltpu.sync_copy(s1v, slots.at[pl.ds(base, TPW)])
    pltpu.sync_copy(s2v, slots.at[pl.ds(S + base, TPW)])

    @pl.when(wid == 0)
    def _tile0():
        tbs = [jnp.sum(jnp.where(lane == e, tb, 0)) for e in range(E)]
        nts = [jnp.sum(jnp.where(lane == e, nt, 0)) for e in range(E)]
        for c in range(NTP // L):
            tv = lane + c * L
            te = zi
            va = zi
            for e in range(E):
                inr = jnp.logical_and(tv >= tbs[e], tv < tbs[e] + nts[e])
                te = jnp.where(inr, e, te)
                va = jnp.where(inr, 1, va)
            tmv[pl.ds(c * L, L)] = te
            tmv[pl.ds(NTP + c * L, L)] = va
        pltpu.sync_copy(tmv, tmeta)
        pltpu.sync_copy(psg, psv)

        def _sum(w, acc):
            return acc + psv[pl.ds(w * L, L)]

        pst = lax.fori_loop(0, NW, _sum, jnp.zeros((L,), jnp.float32))
        a = jnp.sum(pst * tot.astype(jnp.float32)) * (1.0 / (S * S))
        auxs[...] = jnp.full((L,), a, jnp.float32)
        pltpu.sync_copy(auxs, aux)


def _b_body(m_ref, xs_ref, wg_ref, wu_ref, wd_ref, os_ref):
    i = pl.program_id(0)

    @pl.when(m_ref[1, i] == 1)
    def _compute():
        xv = xs_ref[...]
        g = jnp.dot(xv, wg_ref[0], preferred_element_type=jnp.float32)
        u = jnp.dot(xv, wu_ref[0], preferred_element_type=jnp.float32)
        h = g * (1.0 / (1.0 + jnp.exp(-g))) * u
        os_ref[...] = jnp.dot(h, wd_ref[0], preferred_element_type=jnp.float32)


def _sc2_body(osr, slots, ybuf, sv, rows, sem):
    wid = lax.axis_index("s") * NC + lax.axis_index("c")
    base = wid * TPW
    for k in range(K):
        pltpu.sync_copy(slots.at[pl.ds(k * S + base, TPW)], sv)
        pltpu.async_copy(osr.at[sv], rows, sem).wait()
        pltpu.sync_copy(rows, ybuf.at[pl.ds(k * S + base, TPW)])


def _c_body(x1_ref, y0_ref, y1_ref, w0_ref, w1_ref, n3_ref, xo_ref):
    y = (x1_ref[...] + y0_ref[...] * w0_ref[0] + y1_ref[...] * w1_ref[0])
    xo_ref[...] = _rms(y, n3_ref[...])


def kernel(x, Wq, Wk, Wv, Wo, gate_w, Wg, Wu, Wd, norm1, norm2, norm3):
    xf = x.reshape(S, D)
    n1 = norm1.reshape(1, D)
    n2 = norm2.reshape(1, D)
    n3 = norm3.reshape(1, D)

    full = lambda shp: pl.BlockSpec(shp, lambda *_: tuple(0 for _ in shp))

    q, k, v = pl.pallas_call(
        _a1_body,
        grid=(S_BLKS,),
        in_specs=[pl.BlockSpec((S_BLK, D), lambda i: (i, 0)),
                  full((1, D)), full((D, D)), full((D, D)), full((D, D))],
        out_specs=[pl.BlockSpec((S_BLK, D), lambda i: (i, 0))] * 3,
        out_shape=[jax.ShapeDtypeStruct((S, D), jnp.float32)] * 3,
    )(xf, n1, Wq, Wk, Wv)

    attn = pl.pallas_call(
        _a2_body,
        grid=(NH // 2, Q_BLKS),
        in_specs=[pl.BlockSpec((Q_BLK, 2 * HD), lambda h, i: (i, h)),
                  pl.BlockSpec((S, 2 * HD), lambda h, i: (0, h)),
                  pl.BlockSpec((S, 2 * HD), lambda h, i: (0, h))],
        out_specs=pl.BlockSpec((Q_BLK, 2 * HD), lambda h, i: (i, h)),
        out_shape=jax.ShapeDtypeStruct((S, D), jnp.float32),
    )(q, k, v)

    x1, hx2, logt = pl.pallas_call(
        _a3_body,
        grid=(S_BLKS,),
        in_specs=[pl.BlockSpec((S_BLK, D), lambda i: (i, 0)),
                  pl.BlockSpec((S_BLK, D), lambda i: (i, 0)),
                  full((D, D)), full((1, D)), full((D, E))],
        out_specs=[pl.BlockSpec((S_BLK, D), lambda i: (i, 0)),
                   pl.BlockSpec((S_BLK, D), lambda i: (i, 0)),
                   pl.BlockSpec((E, S_BLK), lambda i: (0, i))],
        out_shape=[jax.ShapeDtypeStruct((S, D), jnp.float32),
                   jax.ShapeDtypeStruct((S, D), jnp.float32),
                   jax.ShapeDtypeStruct((E, S), jnp.float32)],
    )(attn, xf, Wo, n2, gate_w)

    mesh = plsc.VectorSubcoreMesh(core_axis_name="c", subcore_axis_name="s",
                                  num_cores=NC, num_subcores=NS)
    sc_params = pltpu.CompilerParams(needs_layout_passes=False)

    sc1a = pl.kernel(
        _sc1a_body,
        out_type=[jax.ShapeDtypeStruct((K * S,), jnp.int32),
                  jax.ShapeDtypeStruct((K * S,), jnp.float32),
                  jax.ShapeDtypeStruct((NW * L,), jnp.int32),
                  jax.ShapeDtypeStruct((NW * L,), jnp.float32)],
        mesh=mesh,
        scratch_types=[pltpu.VMEM((E * TPW,), jnp.float32),
                       pltpu.VMEM((TPW,), jnp.int32),
                       pltpu.VMEM((TPW,), jnp.int32),
                       pltpu.VMEM((TPW,), jnp.float32),
                       pltpu.VMEM((TPW,), jnp.float32),
                       pltpu.VMEM((L,), jnp.int32),
                       pltpu.VMEM((L,), jnp.float32)],
        compiler_params=sc_params,
    )
    epair, wpair, lcg, psg = sc1a(logt)

    sc1b = pl.kernel(
        _sc1b_body,
        out_type=[jax.ShapeDtypeStruct((P, D), jnp.float32),
                  jax.ShapeDtypeStruct((K * S,), jnp.int32),
                  jax.ShapeDtypeStruct((2 * NTP,), jnp.int32),
                  jax.ShapeDtypeStruct((L,), jnp.float32)],
        mesh=mesh,
        scratch_types=[pltpu.VMEM((NW * L,), jnp.int32),
                       pltpu.VMEM((NW * L,), jnp.float32),
                       pltpu.VMEM((TPW,), jnp.int32),
                       pltpu.VMEM((TPW,), jnp.int32),
                       pltpu.VMEM((TPW,), jnp.int32),
                       pltpu.VMEM((TPW,), jnp.int32),
                       pltpu.VMEM((TPW, D), jnp.float32),
                       pltpu.VMEM((2 * NTP,), jnp.int32),
                       pltpu.VMEM((L,), jnp.float32),
                       pltpu.SemaphoreType.DMA],
        compiler_params=sc_params,
    )
    xs, slots, tmeta, aux = sc1b(epair, lcg, psg, hx2)

    osr = pl.pallas_call(
        _b_body,
        grid_spec=pltpu.PrefetchScalarGridSpec(
            num_scalar_prefetch=1,
            grid=(NT,),
            in_specs=[pl.BlockSpec((TS, D), lambda i, m: (i, 0)),
                      pl.BlockSpec((1, D, F), lambda i, m: (m[0, i], 0, 0)),
                      pl.BlockSpec((1, D, F), lambda i, m: (m[0, i], 0, 0)),
                      pl.BlockSpec((1, F, D), lambda i, m: (m[0, i], 0, 0))],
            out_specs=pl.BlockSpec((TS, D), lambda i, m: (i, 0)),
        ),
        out_shape=jax.ShapeDtypeStruct((P, D), jnp.float32),
    )(tmeta.reshape(2, NTP), xs, Wg, Wu, Wd)

    sc2 = pl.kernel(
        _sc2_body,
        out_type=jax.ShapeDtypeStruct((K * S, D), jnp.float32),
        mesh=mesh,
        scratch_types=[pltpu.VMEM((TPW,), jnp.int32),
                       pltpu.VMEM((TPW, D), jnp.float32),
                       pltpu.SemaphoreType.DMA],
        compiler_params=sc_params,
    )
    ybuf = sc2(osr, slots)

    wp3 = wpair.reshape(K, S, 1)
    xo = pl.pallas_call(
        _c_body,
        grid=(S_BLKS,),
        in_specs=[pl.BlockSpec((S_BLK, D), lambda i: (i, 0)),
                  pl.BlockSpec((S_BLK, D), lambda i: (i, 0)),
                  pl.BlockSpec((S_BLK, D), lambda i: (i + S_BLKS, 0)),
                  pl.BlockSpec((1, S_BLK, 1), lambda i: (0, i, 0)),
                  pl.BlockSpec((1, S_BLK, 1), lambda i: (1, i, 0)),
                  full((1, D))],
        out_specs=pl.BlockSpec((S_BLK, D), lambda i: (i, 0)),
        out_shape=jax.ShapeDtypeStruct((S, D), jnp.float32),
    )(x1, ybuf, ybuf, wp3, wp3, n3)

    return xo.reshape(B, S, D), aux[0]
